# Initial kernel scaffold; baseline (speedup 1.0000x reference)
import jax
import jax.numpy as jnp
from jax.experimental import pallas as pl


def kernel(features, cluster_assignments, cluster_centers):
    raise NotImplementedError("write your pallas kernel here")


# speedup vs baseline: 22.2681x; 22.2681x over previous
import functools

import jax
import jax.numpy as jnp
from jax import lax
from jax.experimental import pallas as pl

_N = 4096
_D = 128
_K = 256
_BN = 512
_GRID = _N // _BN
_THRESH = 1.0


def _tc_body(f_ref, c_ref, a_ref, o_ref):
    i = pl.program_id(0)
    f = f_ref[...]
    c = c_ref[...]
    g = lax.dot_general(f, c, (((1,), (1,)), ((), ())),
                        preferred_element_type=jnp.float32)
    f2 = jnp.sum(f * f, axis=1, keepdims=True)
    c2 = jnp.sum(c * c, axis=1)[None, :]
    d2 = jnp.maximum(f2 + c2 - 2.0 * g, 0.0)
    d = jnp.sqrt(d2)
    a = a_ref[0, 0, :]
    cols = lax.broadcasted_iota(jnp.int32, (_BN, _K), 1)
    assigned = cols == a[:, None]
    inf = jnp.float32(jnp.inf)
    term_other = jnp.sum(jnp.where(assigned, inf,
                                   jnp.maximum(d - _THRESH, 0.0)))
    d_assigned = jnp.sum(jnp.where(assigned, d, 0.0), axis=1)
    term_assigned = jnp.sum(jnp.maximum(_THRESH - d_assigned, 0.0))

    part = (term_assigned + term_other).reshape(1, 1)

    @pl.when(i == 0)
    def _init():
        o_ref[...] = jnp.zeros((1, 1), jnp.float32)

    o_ref[...] += part

    @pl.when(i == pl.num_programs(0) - 1)
    def _finish():
        o_ref[...] = o_ref[...] / jnp.float32(_N)


@functools.partial(jax.jit, static_argnames=("interpret",))
def _run(features, assignments_i32, cluster_centers, interpret=False):
    a3 = assignments_i32.reshape(_GRID, 1, _BN)
    out = pl.pallas_call(
        _tc_body,
        grid=(_GRID,),
        in_specs=[
            pl.BlockSpec((_BN, _D), lambda i: (i, 0)),
            pl.BlockSpec((_K, _D), lambda i: (0, 0)),
            pl.BlockSpec((1, 1, _BN), lambda i: (i, 0, 0)),
        ],
        out_specs=pl.BlockSpec((1, 1), lambda i: (0, 0)),
        out_shape=jax.ShapeDtypeStruct((1, 1), jnp.float32),
        interpret=interpret,
    )(features, cluster_centers, a3)
    return out[0, 0]


def kernel(features, cluster_assignments, cluster_centers):
    return _run(features, cluster_assignments.astype(jnp.int32),
                cluster_centers)
